# Initial kernel scaffold; baseline (speedup 1.0000x reference)
import jax
import jax.numpy as jnp
from jax.experimental import pallas as pl


def kernel(x, gw1, gb1, gw2, gb2, w1, b1, w2, b2):
    raise NotImplementedError("write your pallas kernel here")


# speedup vs baseline: 1.5165x; 1.5165x over previous
import functools

import jax
import jax.numpy as jnp
from jax.experimental import pallas as pl
from jax.experimental.pallas import tpu as pltpu

DIM = 768
HID = 4 * DIM
GDIM = 2 * DIM
NUM_EXPERTS = 8
TOP_K = 2
NTOK = 2048
BT = 256
NT = NTOK // BT

_NEG = -3.0e38


def _silu(v):
    return v * jax.lax.logistic(v)


def _top2_coef(g):
    e_idx = jax.lax.broadcasted_iota(jnp.int32, g.shape, 1)
    max1 = jnp.max(g, axis=1, keepdims=True)
    i1 = jnp.min(jnp.where(g == max1, e_idx, NUM_EXPERTS), axis=1, keepdims=True)
    g2 = jnp.where(e_idx == i1, _NEG, g)
    max2 = jnp.max(g2, axis=1, keepdims=True)
    i2 = jnp.min(jnp.where(g2 == max2, e_idx, NUM_EXPERTS), axis=1, keepdims=True)
    p = jnp.exp(max2 - max1)
    wa = 1.0 / (1.0 + p)
    wb = 1.0 - wa
    sel1 = (e_idx == i1).astype(jnp.float32)
    sel2 = (e_idx == i2).astype(jnp.float32)
    return wa * sel1 + wb * sel2


def _moe_body(x_ref, gw1_ref, gb1_ref, gw2_ref, gb2_ref, w1_ref, b1_ref,
              w2_ref, b2_ref, out_ref, h_s, coef_s, acc_s):
    e = pl.program_id(0)
    t = pl.program_id(1)
    row = pl.ds(t * BT, BT)

    @pl.when(e == 0)
    def _gate_and_hidden():
        xt = x_ref[...]
        hg = _silu(jnp.dot(xt, gw1_ref[...],
                           preferred_element_type=jnp.float32) + gb1_ref[...])
        g = jnp.dot(hg, gw2_ref[...],
                    preferred_element_type=jnp.float32) + gb2_ref[...]
        coef = _top2_coef(g)
        coef_s[row, :] = coef
        h = _silu(jnp.dot(xt, w1_ref[...],
                          preferred_element_type=jnp.float32) + b1_ref[...])
        h_s[row, :] = h.astype(jnp.bfloat16)
        acc_s[row, :] = jnp.dot(coef, b2_ref[...],
                                preferred_element_type=jnp.float32)

    coef_t = coef_s[row, :]
    sel = (jax.lax.broadcasted_iota(jnp.int32, (BT, NUM_EXPERTS), 1) == e)
    ce = jnp.sum(jnp.where(sel, coef_t, 0.0), axis=1, keepdims=True)
    h = h_s[row, :].astype(jnp.float32)
    part = jnp.dot(h, w2_ref[0], preferred_element_type=jnp.float32)
    acc_s[row, :] += ce * part

    @pl.when(e == NUM_EXPERTS - 1)
    def _write_out():
        out_ref[...] = acc_s[row, :]


@jax.jit
def kernel(x, gw1, gb1, gw2, gb2, w1, b1, w2, b2):
    grid = (NUM_EXPERTS, NT)
    out = pl.pallas_call(
        _moe_body,
        grid=grid,
        in_specs=[
            pl.BlockSpec((BT, DIM), lambda e, t: (t, 0)),
            pl.BlockSpec((DIM, GDIM), lambda e, t: (0, 0)),
            pl.BlockSpec((GDIM,), lambda e, t: (0,)),
            pl.BlockSpec((GDIM, NUM_EXPERTS), lambda e, t: (0, 0)),
            pl.BlockSpec((NUM_EXPERTS,), lambda e, t: (0,)),
            pl.BlockSpec((DIM, HID), lambda e, t: (0, 0)),
            pl.BlockSpec((HID,), lambda e, t: (0,)),
            pl.BlockSpec((1, HID, DIM), lambda e, t: (e, 0, 0)),
            pl.BlockSpec((NUM_EXPERTS, DIM), lambda e, t: (0, 0)),
        ],
        out_specs=pl.BlockSpec((BT, DIM), lambda e, t: (t, 0)),
        out_shape=jax.ShapeDtypeStruct((NTOK, DIM), jnp.float32),
        scratch_shapes=[
            pltpu.VMEM((NTOK, HID), jnp.bfloat16),
            pltpu.VMEM((NTOK, NUM_EXPERTS), jnp.float32),
            pltpu.VMEM((NTOK, DIM), jnp.float32),
        ],
        compiler_params=pltpu.CompilerParams(
            dimension_semantics=("arbitrary", "arbitrary"),
        ),
    )(x, gw1, gb1, gw2, gb2, w1, b1, w2, b2)
    return out
